# baseline (speedup 1.0000x reference)
import functools

import jax
import jax.numpy as jnp
from jax import lax
from jax.experimental import pallas as pl
from jax.experimental.pallas import tpu as pltpu
from jax.experimental.pallas import tpu_sc as plsc

EPS = 1e-5
BLK = 128
SUP = 25
CSUP = 13
ROW_BLK = 1024


def _sc_segment_sums(edge_index, edge_attr, n):
    e, d = edge_attr.shape
    info = plsc.get_sparse_core_info()
    nc, ns = info.num_cores, info.num_subcores
    nw = nc * ns
    nblocks = e // BLK
    rng = nblocks // 4
    n_sup = rng // SUP
    cnt_base = nblocks // nw
    cnt_extra = nblocks - cnt_base * nw
    n_csup = cnt_base // CSUP

    idx3 = edge_index.reshape(2, nblocks, BLK).transpose(1, 0, 2)
    attr4 = edge_attr.T.reshape(d // 8, 8, nblocks, BLK).transpose(0, 2, 1, 3)

    mesh = plsc.VectorSubcoreMesh(core_axis_name="c", subcore_axis_name="s")

    @functools.partial(
        pl.kernel,
        out_type=(
            jax.ShapeDtypeStruct((4, d, n), jnp.float32),
            jax.ShapeDtypeStruct((nc, ns, n), jnp.float32),
        ),
        mesh=mesh,
        compiler_params=pltpu.CompilerParams(use_tc_tiling_on_sc=False, needs_layout_passes=False),
        scratch_types=[
            pltpu.VMEM((SUP, BLK), jnp.int32),
            pltpu.VMEM((SUP, BLK), jnp.int32),
            pltpu.VMEM((SUP, 2, BLK), jnp.float32),
            pltpu.VMEM((SUP, 2, BLK), jnp.float32),
            pltpu.VMEM((CSUP, BLK), jnp.int32),
            pltpu.VMEM((CSUP, BLK), jnp.int32),
            pltpu.VMEM((4, n), jnp.float32),
            pltpu.VMEM((2, n), jnp.float32),
            pltpu.SemaphoreType.DMA,
            pltpu.SemaphoreType.DMA,
            pltpu.SemaphoreType.DMA,
            pltpu.SemaphoreType.DMA,
            pltpu.SemaphoreType.DMA,
            pltpu.SemaphoreType.DMA,
        ],
    )
    def seg_kernel(idx_hbm, attr_hbm, val_out, cnt_out,
                   idx_v0, idx_v1, av_v0, av_v1, cidx_v0, cidx_v1,
                   planes, cplanes,
                   sem_i0, sem_i1, sem_a0, sem_a1, sem_c0, sem_c1):
        c = lax.axis_index("c")
        s = lax.axis_index("s")
        wid = c * ns + s
        fg = wid % 8
        bg = wid // 8
        f0 = 2 * fg
        f_hi = f0 // 8
        lo0 = f0 % 8
        ones = jnp.ones((16,), jnp.float32)

        def zf(i, _):
            z = jnp.zeros((16,), jnp.float32)
            for k in range(4):
                planes[k, pl.ds(i * 16, 16)] = z
            for k in range(2):
                cplanes[k, pl.ds(i * 16, 16)] = z
            return 0
        lax.fori_loop(0, n // 16, zf, 0)

        base0 = bg * rng

        def _start(t, idx_b, av_b, sem_i, sem_a):
            base = base0 + t * SUP
            pltpu.async_copy(idx_hbm.at[pl.ds(base, SUP), 1], idx_b, sem_i)
            pltpu.async_copy(
                attr_hbm.at[f_hi, pl.ds(base, SUP), pl.ds(lo0, 2)],
                av_b, sem_a)

        def _wait(t, idx_b, av_b, sem_i, sem_a):
            base = base0 + t * SUP
            pltpu.make_async_copy(idx_hbm.at[pl.ds(base, SUP), 1],
                                  idx_b, sem_i).wait()
            pltpu.make_async_copy(
                attr_hbm.at[f_hi, pl.ds(base, SUP), pl.ds(lo0, 2)],
                av_b, sem_a).wait()

        def _process(idx_b, av_b):
            nj = BLK // 16
            for b in range(SUP):
                ivs = [idx_b[b, pl.ds(j * 16, 16)] for j in range(nj)]
                for j in range(nj):
                    for ff in range(2):
                        vv = av_b[b, ff, pl.ds(j * 16, 16)]
                        plsc.addupdate_scatter(
                            planes.at[2 * ff + (j % 2)], [ivs[j]], vv)

        pltpu.sync_copy(idx_hbm.at[pl.ds(base0, SUP), 1], idx_v0)
        pltpu.sync_copy(attr_hbm.at[f_hi, pl.ds(base0, SUP), pl.ds(lo0, 2)],
                        av_v0)
        _process(idx_v0, av_v0)
        _start(1, idx_v0, av_v0, sem_i0, sem_a0)
        _start(2, idx_v1, av_v1, sem_i1, sem_a1)

        def pair_step(i, _):
            t0 = 2 * i + 1
            _wait(t0, idx_v0, av_v0, sem_i0, sem_a0)
            _process(idx_v0, av_v0)

            @pl.when(t0 + 2 < n_sup)
            def _():
                _start(t0 + 2, idx_v0, av_v0, sem_i0, sem_a0)
            _wait(t0 + 1, idx_v1, av_v1, sem_i1, sem_a1)
            _process(idx_v1, av_v1)

            @pl.when(t0 + 3 < n_sup)
            def _():
                _start(t0 + 3, idx_v1, av_v1, sem_i1, sem_a1)
            return 0
        lax.fori_loop(0, (n_sup - 1) // 2, pair_step, 0)

        def vmerge(i, _):
            sl = pl.ds(i * 16, 16)
            planes[0, sl] = planes[0, sl] + planes[1, sl]
            planes[2, sl] = planes[2, sl] + planes[3, sl]
            return 0
        lax.fori_loop(0, n // 16, vmerge, 0)

        cbase = wid * cnt_base + jnp.minimum(wid, cnt_extra)

        def _cstart(t, buf, sem):
            pltpu.async_copy(idx_hbm.at[pl.ds(cbase + t * CSUP, CSUP), 1],
                             buf, sem)

        def _cwait(t, buf, sem):
            pltpu.make_async_copy(idx_hbm.at[pl.ds(cbase + t * CSUP, CSUP), 1],
                                  buf, sem).wait()

        def _cprocess(buf):
            nj = BLK // 16
            for b in range(CSUP):
                ivs = [buf[b, pl.ds(j * 16, 16)] for j in range(nj)]
                for j in range(nj):
                    plsc.addupdate_scatter(cplanes.at[j % 2], [ivs[j]], ones)

        _cstart(0, cidx_v0, sem_c0)
        _cstart(1, cidx_v1, sem_c1)

        def cpair_step(i, _):
            t0 = 2 * i
            _cwait(t0, cidx_v0, sem_c0)
            _cprocess(cidx_v0)

            @pl.when(t0 + 2 < n_csup)
            def _():
                _cstart(t0 + 2, cidx_v0, sem_c0)
            _cwait(t0 + 1, cidx_v1, sem_c1)
            _cprocess(cidx_v1)

            @pl.when(t0 + 3 < n_csup)
            def _():
                _cstart(t0 + 3, cidx_v1, sem_c1)
            return 0
        lax.fori_loop(0, n_csup // 2, cpair_step, 0)
        if cnt_extra:
            @pl.when(wid < cnt_extra)
            def _():
                b0 = cbase + cnt_base
                pltpu.sync_copy(idx_hbm.at[pl.ds(b0, 1), 1],
                                cidx_v0.at[pl.ds(0, 1)])
                for j in range(BLK // 16):
                    iv = cidx_v0[0, pl.ds(j * 16, 16)]
                    plsc.addupdate_scatter(cplanes.at[0], [iv], ones)

        def merge(i, _):
            sl = pl.ds(i * 16, 16)
            planes[0, sl] = ((planes[0, sl] + planes[1, sl])
                             + (planes[2, sl] + planes[3, sl]))
            cplanes[0, sl] = cplanes[0, sl] + cplanes[1, sl]
            return 0
        lax.fori_loop(0, n // 16, merge, 0)

        pltpu.sync_copy(planes.at[0], val_out.at[bg, f0])
        pltpu.sync_copy(planes.at[2], val_out.at[bg, f0 + 1])
        pltpu.sync_copy(cplanes.at[0], cnt_out.at[c, s])

    return seg_kernel(idx3, attr4)


def _mlp1_body(x_ref, b_ref, u_ref, w1x_ref, w1u_ref, o_ref):
    g = u_ref.shape[0]
    x = x_ref[...]
    bb = b_ref[...]
    oh = (bb == lax.broadcasted_iota(jnp.int32, (bb.shape[0], g), 1)
          ).astype(jnp.float32)
    uw = jnp.dot(u_ref[...], w1u_ref[...], preferred_element_type=jnp.float32)
    o_ref[...] = (jnp.dot(x, w1x_ref[...], preferred_element_type=jnp.float32)
                  + jnp.dot(oh, uw, preferred_element_type=jnp.float32))


def _mlp2_body(z_ref, s_ref, c_ref, w1p_ref, w2_ref,
               a1_ref, c1_ref, a2_ref, c2_ref, o_ref):
    v = (s_ref[0] + s_ref[1]) + (s_ref[2] + s_ref[3])
    ctot = jnp.maximum((c_ref[0] + c_ref[1]).sum(axis=0, keepdims=True), 1.0)
    vdiv = v / ctot
    pp = lax.dot_general(vdiv, w1p_ref[...], (((0,), (0,)), ((), ())),
                         preferred_element_type=jnp.float32)
    z1 = z_ref[...] + pp
    h = jnp.maximum(z1 * a1_ref[...] + c1_ref[...], 0.0)
    z2 = jnp.dot(h, w2_ref[...], preferred_element_type=jnp.float32)
    o_ref[...] = z2 * a2_ref[...] + c2_ref[...]


def kernel(x, edge_index, edge_attr, u, batch, W1, b1, g1, bt1, rm1, rv1,
           W2, b2, g2, bt2, rm2, rv2):
    n, x_ind = x.shape
    e, d = edge_attr.shape
    g, u_ind = u.shape
    hs = W1.shape[0]

    vals, cnts = _sc_segment_sums(edge_index, edge_attr, n)

    a1 = g1 / jnp.sqrt(rv1 + EPS)
    c1 = (b1 - rm1) * a1 + bt1
    a2 = g2 / jnp.sqrt(rv2 + EPS)
    c2 = (b2 - rm2) * a2 + bt2

    w1x = W1[:, :x_ind].T
    w1u = W1[:, x_ind:x_ind + u_ind].T
    w1p = W1[:, x_ind + u_ind:].T
    w2 = W2.T
    batch2d = batch.reshape(n, 1)

    grid = (n + ROW_BLK - 1) // ROW_BLK
    z1x = pl.pallas_call(
        _mlp1_body,
        grid=(grid,),
        in_specs=[
            pl.BlockSpec((ROW_BLK, x_ind), lambda i: (i, 0)),
            pl.BlockSpec((ROW_BLK, 1), lambda i: (i, 0)),
            pl.BlockSpec((g, u_ind), lambda i: (0, 0)),
            pl.BlockSpec((x_ind, hs), lambda i: (0, 0)),
            pl.BlockSpec((u_ind, hs), lambda i: (0, 0)),
        ],
        out_specs=pl.BlockSpec((ROW_BLK, hs), lambda i: (i, 0)),
        out_shape=jax.ShapeDtypeStruct((n, hs), jnp.float32),
    )(x, batch2d, u, w1x, w1u)
    out = pl.pallas_call(
        _mlp2_body,
        grid=(grid,),
        in_specs=[
            pl.BlockSpec((ROW_BLK, hs), lambda i: (i, 0)),
            pl.BlockSpec((4, d, ROW_BLK), lambda i: (0, 0, i)),
            pl.BlockSpec((2, d, ROW_BLK), lambda i: (0, 0, i)),
            pl.BlockSpec((d, hs), lambda i: (0, 0)),
            pl.BlockSpec((hs, hs), lambda i: (0, 0)),
            pl.BlockSpec((1, hs), lambda i: (0, 0)),
            pl.BlockSpec((1, hs), lambda i: (0, 0)),
            pl.BlockSpec((1, hs), lambda i: (0, 0)),
            pl.BlockSpec((1, hs), lambda i: (0, 0)),
        ],
        out_specs=pl.BlockSpec((ROW_BLK, hs), lambda i: (i, 0)),
        out_shape=jax.ShapeDtypeStruct((n, hs), jnp.float32),
    )(z1x, vals, cnts, w1p, w2,
      a1.reshape(1, hs), c1.reshape(1, hs), a2.reshape(1, hs), c2.reshape(1, hs))
    return out


# speedup vs baseline: 1.2521x; 1.2521x over previous
import functools

import jax
import jax.numpy as jnp
from jax import lax
from jax.experimental import pallas as pl
from jax.experimental.pallas import tpu as pltpu
from jax.experimental.pallas import tpu_sc as plsc

EPS = 1e-5
BLK = 128
SUP = 25
CSUP = 13
ROW_BLK = 1024


def _sc_segment_sums(edge_index, edge_attr, n):
    e, d = edge_attr.shape
    info = plsc.get_sparse_core_info()
    nc, ns = info.num_cores, info.num_subcores
    nblocks = e // BLK
    half_blocks = nblocks // 2
    n_sup = half_blocks // SUP
    cnt_base = half_blocks // ns
    cnt_extra = half_blocks - cnt_base * ns
    n_csup = cnt_base // CSUP
    cnt_tail = cnt_base - n_csup * CSUP

    idx3 = edge_index.reshape(2, nblocks, BLK).transpose(1, 0, 2)
    attr4 = edge_attr.T.reshape(d // 8, 8, nblocks, BLK).transpose(0, 2, 1, 3)

    mesh = plsc.VectorSubcoreMesh(core_axis_name="c", subcore_axis_name="s")

    @functools.partial(
        pl.kernel,
        out_type=(
            jax.ShapeDtypeStruct((2, d, n), jnp.float32),
            jax.ShapeDtypeStruct((2, d, n), jnp.float32),
        ),
        mesh=mesh,
        compiler_params=pltpu.CompilerParams(use_tc_tiling_on_sc=False, needs_layout_passes=False),
        scratch_types=[
            pltpu.VMEM((SUP, BLK), jnp.int32),
            pltpu.VMEM((SUP, BLK), jnp.int32),
            pltpu.VMEM((SUP, BLK), jnp.float32),
            pltpu.VMEM((SUP, BLK), jnp.float32),
            pltpu.VMEM((CSUP, BLK), jnp.int32),
            pltpu.VMEM((CSUP, BLK), jnp.int32),
            pltpu.VMEM((4, n), jnp.float32),
            pltpu.VMEM((2, n), jnp.float32),
            pltpu.SemaphoreType.DMA,
            pltpu.SemaphoreType.DMA,
            pltpu.SemaphoreType.DMA,
            pltpu.SemaphoreType.DMA,
            pltpu.SemaphoreType.DMA,
            pltpu.SemaphoreType.DMA,
        ],
    )
    def seg_kernel(idx_hbm, attr_hbm, val_out, cnt_out,
                   idx_v0, idx_v1, av_v0, av_v1, cidx_v0, cidx_v1,
                   planes, cplanes,
                   sem_i0, sem_i1, sem_a0, sem_a1, sem_c0, sem_c1):
        c = lax.axis_index("c")
        s = lax.axis_index("s")
        wid = c * ns + s
        f = wid % d
        h = wid // d
        f_hi = f // 8
        f_lo = f % 8
        ones = jnp.ones((16,), jnp.float32)

        def zf(i, _):
            z = jnp.zeros((16,), jnp.float32)
            for k in range(4):
                planes[k, pl.ds(i * 16, 16)] = z
            for k in range(2):
                cplanes[k, pl.ds(i * 16, 16)] = z
            return 0
        lax.fori_loop(0, n // 16, zf, 0)

        base0 = h * half_blocks

        def _start(t, idx_b, av_b, sem_i, sem_a):
            base = base0 + t * SUP
            pltpu.async_copy(idx_hbm.at[pl.ds(base, SUP), 1], idx_b, sem_i)
            pltpu.async_copy(attr_hbm.at[f_hi, pl.ds(base, SUP), f_lo],
                             av_b, sem_a)

        def _wait(t, idx_b, av_b, sem_i, sem_a):
            base = base0 + t * SUP
            pltpu.make_async_copy(idx_hbm.at[pl.ds(base, SUP), 1],
                                  idx_b, sem_i).wait()
            pltpu.make_async_copy(attr_hbm.at[f_hi, pl.ds(base, SUP), f_lo],
                                  av_b, sem_a).wait()

        def _process(idx_b, av_b):
            nj = BLK // 16
            for b in range(SUP):
                ivs = [idx_b[b, pl.ds(j * 16, 16)] for j in range(nj)]
                vvs = [av_b[b, pl.ds(j * 16, 16)] for j in range(nj)]
                for j in range(nj):
                    plsc.addupdate_scatter(planes.at[j % 4], [ivs[j]], vvs[j])

        _start(0, idx_v0, av_v0, sem_i0, sem_a0)
        _start(1, idx_v1, av_v1, sem_i1, sem_a1)

        def pair_step(i, _):
            t0 = 2 * i
            _wait(t0, idx_v0, av_v0, sem_i0, sem_a0)
            _process(idx_v0, av_v0)

            @pl.when(t0 + 2 < n_sup)
            def _():
                _start(t0 + 2, idx_v0, av_v0, sem_i0, sem_a0)
            _wait(t0 + 1, idx_v1, av_v1, sem_i1, sem_a1)
            _process(idx_v1, av_v1)

            @pl.when(t0 + 3 < n_sup)
            def _():
                _start(t0 + 3, idx_v1, av_v1, sem_i1, sem_a1)
            return 0
        lax.fori_loop(0, n_sup // 2, pair_step, 0)

        cbase = base0 + s * cnt_base + jnp.minimum(s, cnt_extra)

        def _cstart(t, buf, sem):
            pltpu.async_copy(idx_hbm.at[pl.ds(cbase + t * CSUP, CSUP), 1],
                             buf, sem)

        def _cwait(t, buf, sem):
            pltpu.make_async_copy(idx_hbm.at[pl.ds(cbase + t * CSUP, CSUP), 1],
                                  buf, sem).wait()

        def _cprocess(buf):
            nj = BLK // 16
            for b in range(CSUP):
                ivs = [buf[b, pl.ds(j * 16, 16)] for j in range(nj)]
                for j in range(nj):
                    plsc.addupdate_scatter(cplanes.at[j % 2], [ivs[j]], ones)

        _cstart(0, cidx_v0, sem_c0)
        _cstart(1, cidx_v1, sem_c1)

        def cpair_step(i, _):
            t0 = 2 * i
            _cwait(t0, cidx_v0, sem_c0)
            _cprocess(cidx_v0)

            @pl.when(t0 + 2 < n_csup)
            def _():
                _cstart(t0 + 2, cidx_v0, sem_c0)
            _cwait(t0 + 1, cidx_v1, sem_c1)
            _cprocess(cidx_v1)

            @pl.when(t0 + 3 < n_csup)
            def _():
                _cstart(t0 + 3, cidx_v1, sem_c1)
            return 0
        lax.fori_loop(0, n_csup // 2, cpair_step, 0)
        if cnt_tail:
            b0 = cbase + n_csup * CSUP
            pltpu.sync_copy(idx_hbm.at[pl.ds(b0, cnt_tail), 1],
                            cidx_v0.at[pl.ds(0, cnt_tail)])
            for b in range(cnt_tail):
                for j in range(BLK // 16):
                    iv = cidx_v0[b, pl.ds(j * 16, 16)]
                    plsc.addupdate_scatter(cplanes.at[0], [iv], ones)
        if cnt_extra:
            @pl.when(s < cnt_extra)
            def _():
                b0 = cbase + cnt_base
                pltpu.sync_copy(idx_hbm.at[pl.ds(b0, 1), 1],
                                cidx_v0.at[pl.ds(0, 1)])
                for j in range(BLK // 16):
                    iv = cidx_v0[0, pl.ds(j * 16, 16)]
                    plsc.addupdate_scatter(cplanes.at[0], [iv], ones)

        def merge(i, _):
            sl = pl.ds(i * 16, 16)
            planes[0, sl] = ((planes[0, sl] + planes[1, sl])
                             + (planes[2, sl] + planes[3, sl]))
            cplanes[0, sl] = cplanes[0, sl] + cplanes[1, sl]
            return 0
        lax.fori_loop(0, n // 16, merge, 0)

        pltpu.sync_copy(planes.at[0], val_out.at[h, f])
        pltpu.sync_copy(cplanes.at[0], cnt_out.at[h, f])

    return seg_kernel(idx3, attr4)


def _mlp1_body(x_ref, b_ref, u_ref, w1x_ref, w1u_ref, o_ref):
    g = u_ref.shape[0]
    x = x_ref[...]
    bb = b_ref[...]
    oh = (bb == lax.broadcasted_iota(jnp.int32, (bb.shape[0], g), 1)
          ).astype(jnp.float32)
    uw = jnp.dot(u_ref[...], w1u_ref[...], preferred_element_type=jnp.float32)
    o_ref[...] = (jnp.dot(x, w1x_ref[...], preferred_element_type=jnp.float32)
                  + jnp.dot(oh, uw, preferred_element_type=jnp.float32))


def _mlp2_body(z_ref, s_ref, c_ref, w1p_ref, w2_ref,
               a1_ref, c1_ref, a2_ref, c2_ref, o_ref):
    v = s_ref[0] + s_ref[1]
    ctot = jnp.maximum((c_ref[0] + c_ref[1]).sum(axis=0, keepdims=True), 1.0)
    vdiv = v / ctot
    pp = lax.dot_general(vdiv, w1p_ref[...], (((0,), (0,)), ((), ())),
                         preferred_element_type=jnp.float32)
    z1 = z_ref[...] + pp
    h = jnp.maximum(z1 * a1_ref[...] + c1_ref[...], 0.0)
    z2 = jnp.dot(h, w2_ref[...], preferred_element_type=jnp.float32)
    o_ref[...] = z2 * a2_ref[...] + c2_ref[...]


def kernel(x, edge_index, edge_attr, u, batch, W1, b1, g1, bt1, rm1, rv1,
           W2, b2, g2, bt2, rm2, rv2):
    n, x_ind = x.shape
    e, d = edge_attr.shape
    g, u_ind = u.shape
    hs = W1.shape[0]

    vals, cnts = _sc_segment_sums(edge_index, edge_attr, n)

    a1 = g1 / jnp.sqrt(rv1 + EPS)
    c1 = (b1 - rm1) * a1 + bt1
    a2 = g2 / jnp.sqrt(rv2 + EPS)
    c2 = (b2 - rm2) * a2 + bt2

    w1x = W1[:, :x_ind].T
    w1u = W1[:, x_ind:x_ind + u_ind].T
    w1p = W1[:, x_ind + u_ind:].T
    w2 = W2.T
    batch2d = batch.reshape(n, 1)

    grid = (n + ROW_BLK - 1) // ROW_BLK
    z1x = pl.pallas_call(
        _mlp1_body,
        grid=(grid,),
        in_specs=[
            pl.BlockSpec((ROW_BLK, x_ind), lambda i: (i, 0)),
            pl.BlockSpec((ROW_BLK, 1), lambda i: (i, 0)),
            pl.BlockSpec((g, u_ind), lambda i: (0, 0)),
            pl.BlockSpec((x_ind, hs), lambda i: (0, 0)),
            pl.BlockSpec((u_ind, hs), lambda i: (0, 0)),
        ],
        out_specs=pl.BlockSpec((ROW_BLK, hs), lambda i: (i, 0)),
        out_shape=jax.ShapeDtypeStruct((n, hs), jnp.float32),
    )(x, batch2d, u, w1x, w1u)
    out = pl.pallas_call(
        _mlp2_body,
        grid=(grid,),
        in_specs=[
            pl.BlockSpec((ROW_BLK, hs), lambda i: (i, 0)),
            pl.BlockSpec((2, d, ROW_BLK), lambda i: (0, 0, i)),
            pl.BlockSpec((2, d, ROW_BLK), lambda i: (0, 0, i)),
            pl.BlockSpec((d, hs), lambda i: (0, 0)),
            pl.BlockSpec((hs, hs), lambda i: (0, 0)),
            pl.BlockSpec((1, hs), lambda i: (0, 0)),
            pl.BlockSpec((1, hs), lambda i: (0, 0)),
            pl.BlockSpec((1, hs), lambda i: (0, 0)),
            pl.BlockSpec((1, hs), lambda i: (0, 0)),
        ],
        out_specs=pl.BlockSpec((ROW_BLK, hs), lambda i: (i, 0)),
        out_shape=jax.ShapeDtypeStruct((n, hs), jnp.float32),
    )(z1x, vals, cnts, w1p, w2,
      a1.reshape(1, hs), c1.reshape(1, hs), a2.reshape(1, hs), c2.reshape(1, hs))
    return out
